# baseline (speedup 1.0000x reference)
import jax
import jax.numpy as jnp
from jax import lax
from jax.experimental import pallas as pl
from jax.experimental.pallas import tpu as pltpu
from jax.experimental.pallas import tpu_sc as plsc

N = 10000
E = 320000
NHID = 128
NCLASS = 40

NC = 2
NS = 16
NW = NC * NS
CHUNK = 128
NCHUNK = 4 * (-(-E // (4 * NW * CHUNK)))
E_PAD = NW * NCHUNK * CHUNK
ROWS_PER_TILE = 624
TAIL_BASE = NS * ROWS_PER_TILE
TAIL_ROWS = N - TAIL_BASE

LANES = 16
FB = NHID // LANES


def _segsum_body(m_hbm, sd_hbm, w_hbm, out_hbm,
                 acc, w_v, sd, b16, gsem, isem):
    c = lax.axis_index("c")
    s = lax.axis_index("s")
    wid = c * NS + s

    def _zrow(r, _):
        for f in range(FB):
            b16[0][r, pl.ds(f * LANES, LANES)] = jnp.zeros((LANES,),
                                                           jnp.float32)
        return 0
    lax.fori_loop(0, CHUNK, _zrow, 0)
    base = s * ROWS_PER_TILE
    off = 0
    while off < ROWS_PER_TILE:
        nrow = min(CHUNK, ROWS_PER_TILE - off)
        pltpu.sync_copy(b16[0].at[pl.ds(0, nrow)],
                        acc.at[pl.ds(base + off, nrow)])
        off += nrow

    @pl.when(s == 0)
    def _zero_tail():
        pltpu.sync_copy(b16[0].at[pl.ds(0, TAIL_ROWS)],
                        acc.at[pl.ds(TAIL_BASE, TAIL_ROWS)])

    pltpu.sync_copy(w_hbm.at[wid], w_v)

    plsc.subcore_barrier()

    def _fetch_idx(q, j):
        pltpu.async_copy(sd_hbm.at[wid, j], sd[q], isem[q])

    def _wait_idx(q):
        pltpu.make_async_copy(sd_hbm.at[wid, 0], sd[q], isem[q]).wait()

    def _start_gather(r, q):
        pltpu.async_copy(m_hbm.at[sd[q].at[0]], b16[r], gsem[r])

    def _wait_gather(r, q):
        pltpu.make_async_copy(m_hbm.at[sd[q].at[0]], b16[r], gsem[r]).wait()

    def _scale(r, j):
        def _grp(g, _):
            w16 = w_v[j, pl.ds(g * LANES, LANES)]
            for u in range(LANES):
                e = g * LANES + u
                wb = jnp.full((LANES,), w16[u], jnp.float32)
                for f in range(FB):
                    sl = pl.ds(f * LANES, LANES)
                    b16[r][e, sl] = b16[r][e, sl] * wb
            return 0
        lax.fori_loop(0, CHUNK // LANES, _grp, 0)

    G = NCHUNK // 4
    for q in range(3):
        _fetch_idx(q, q)
    _wait_idx(0)
    _start_gather(0, 0)

    def _quad(g, _):
        for p in range(4):
            j = 4 * g + p
            r = p % 2
            q = p
            _wait_gather(r, q)

            @pl.when(j + 3 < NCHUNK)
            def _():
                _fetch_idx((q + 3) % 4, j + 3)

            @pl.when(j + 1 < NCHUNK)
            def _():
                _wait_idx((q + 1) % 4)
                _start_gather(1 - r, (q + 1) % 4)
            _scale(r, j)
            pltpu.sync_copy(b16[r], acc.at[sd[q].at[1]], add=True)
        return 0
    lax.fori_loop(0, G, _quad, 0)

    plsc.subcore_barrier()

    pltpu.sync_copy(acc.at[pl.ds(base, ROWS_PER_TILE)],
                    out_hbm.at[c, pl.ds(base, ROWS_PER_TILE)])

    @pl.when(s == 0)
    def _out_tail():
        pltpu.sync_copy(acc.at[pl.ds(TAIL_BASE, TAIL_ROWS)],
                        out_hbm.at[c, pl.ds(TAIL_BASE, TAIL_ROWS)])


def _segsum(m, sd_p, w_p, name):
    mesh = plsc.VectorSubcoreMesh(core_axis_name="c", subcore_axis_name="s")
    return pl.kernel(
        _segsum_body,
        out_type=jax.ShapeDtypeStruct((NC, N, NHID), jnp.float32),
        mesh=mesh,
        name=name,
        scratch_types=[
            pltpu.VMEM_SHARED((N, NHID), jnp.float32),
            pltpu.VMEM((NCHUNK, CHUNK), jnp.float32),
            [pltpu.VMEM((2, CHUNK), jnp.int32)] * 4,
            [pltpu.VMEM((CHUNK, NHID), jnp.float32)] * 2,
            [pltpu.SemaphoreType.DMA] * 2,
            [pltpu.SemaphoreType.DMA] * 4,
        ],
    )(m, sd_p, w_p)


ROWS_BLK = 1000
GRID = N // ROWS_BLK


def _dot_t(a, b):
    return lax.dot_general(a, b, (((1,), (1,)), ((), ())),
                           preferred_element_type=jnp.float32)


def _dot(a, b):
    return lax.dot_general(a, b, (((1,), (0,)), ((), ())),
                           preferred_element_type=jnp.float32)


def _gru(agg, x, W_ih, b_ih, W_hh, b_hh):
    gi = _dot_t(agg, W_ih) + b_ih
    gh = _dot_t(x, W_hh) + b_hh
    r = jax.nn.sigmoid(gi[:, 0:NHID] + gh[:, 0:NHID])
    z = jax.nn.sigmoid(gi[:, NHID:2 * NHID] + gh[:, NHID:2 * NHID])
    n = jnp.tanh(gi[:, 2 * NHID:] + r * gh[:, 2 * NHID:])
    return (1.0 - z) * n + z * x


def _embed_conv_body(h_ref, We_ref, be_ref, cw_ref, x_ref, m_ref):
    x = _dot_t(h_ref[...], We_ref[...]) + be_ref[...]
    x_ref[...] = x
    m_ref[...] = _dot(x, cw_ref[...])


def _embed_conv(h, W_emb, b_emb, cw0):
    blk = lambda i: (i, 0)
    full = lambda i: (0, 0)
    return pl.pallas_call(
        _embed_conv_body,
        grid=(GRID,),
        in_specs=[
            pl.BlockSpec((ROWS_BLK, NHID), blk),
            pl.BlockSpec((NHID, NHID), full),
            pl.BlockSpec((1, NHID), full),
            pl.BlockSpec((NHID, NHID), full),
        ],
        out_specs=[pl.BlockSpec((ROWS_BLK, NHID), blk),
                   pl.BlockSpec((ROWS_BLK, NHID), blk)],
        out_shape=[jax.ShapeDtypeStruct((N, NHID), jnp.float32),
                   jax.ShapeDtypeStruct((N, NHID), jnp.float32)],
    )(h, W_emb, b_emb, cw0)


def _gru_conv_body(p_ref, x_ref, Wih_ref, bih_ref, Whh_ref, bhh_ref, cw_ref,
                   xo_ref, mo_ref):
    agg = p_ref[0] + p_ref[1]
    xn = _gru(agg, x_ref[...], Wih_ref[...], bih_ref[...],
              Whh_ref[...], bhh_ref[...])
    xo_ref[...] = xn
    mo_ref[...] = _dot(xn, cw_ref[...])


def _gru_conv(p, x, W_ih, b_ih, W_hh, b_hh, cw1):
    blk = lambda i: (i, 0)
    full = lambda i: (0, 0)
    return pl.pallas_call(
        _gru_conv_body,
        grid=(GRID,),
        in_specs=[
            pl.BlockSpec((NC, ROWS_BLK, NHID), lambda i: (0, i, 0)),
            pl.BlockSpec((ROWS_BLK, NHID), blk),
            pl.BlockSpec((3 * NHID, NHID), full),
            pl.BlockSpec((1, 3 * NHID), full),
            pl.BlockSpec((3 * NHID, NHID), full),
            pl.BlockSpec((1, 3 * NHID), full),
            pl.BlockSpec((NHID, NHID), full),
        ],
        out_specs=[pl.BlockSpec((ROWS_BLK, NHID), blk),
                   pl.BlockSpec((ROWS_BLK, NHID), blk)],
        out_shape=[jax.ShapeDtypeStruct((N, NHID), jnp.float32),
                   jax.ShapeDtypeStruct((N, NHID), jnp.float32)],
    )(p, x, W_ih, b_ih, W_hh, b_hh, cw1)


def _gru_head_body(p_ref, x_ref, xin_ref, Wih_ref, bih_ref, Whh_ref, bhh_ref,
                   Wm_ref, bm_ref, o_ref):
    agg = p_ref[0] + p_ref[1]
    xn = _gru(agg, x_ref[...], Wih_ref[...], bih_ref[...],
              Whh_ref[...], bhh_ref[...])
    xr = xin_ref[...] + xn
    logits = _dot_t(xr, Wm_ref[...]) + bm_ref[...]
    shifted = logits - jnp.max(logits, axis=1, keepdims=True)
    o_ref[...] = shifted - jnp.log(
        jnp.sum(jnp.exp(shifted), axis=1, keepdims=True))


def _gru_head(p, x, x_in, W_ih, b_ih, W_hh, b_hh, W_mlp, b_mlp):
    blk = lambda i: (i, 0)
    full = lambda i: (0, 0)
    return pl.pallas_call(
        _gru_head_body,
        grid=(GRID,),
        in_specs=[
            pl.BlockSpec((NC, ROWS_BLK, NHID), lambda i: (0, i, 0)),
            pl.BlockSpec((ROWS_BLK, NHID), blk),
            pl.BlockSpec((ROWS_BLK, NHID), blk),
            pl.BlockSpec((3 * NHID, NHID), full),
            pl.BlockSpec((1, 3 * NHID), full),
            pl.BlockSpec((3 * NHID, NHID), full),
            pl.BlockSpec((1, 3 * NHID), full),
            pl.BlockSpec((NCLASS, NHID), full),
            pl.BlockSpec((1, NCLASS), full),
        ],
        out_specs=pl.BlockSpec((ROWS_BLK, NCLASS), blk),
        out_shape=jax.ShapeDtypeStruct((N, NCLASS), jnp.float32),
    )(p, x, x_in, W_ih, b_ih, W_hh, b_hh, W_mlp, b_mlp)


def kernel(h, edge_index, edge_weight, W_emb, b_emb, conv_w,
           W_ih, b_ih, W_hh, b_hh, W_mlp, b_mlp):
    pad = E_PAD - E
    src_p = jnp.concatenate(
        [edge_index[0], jnp.zeros((pad,), jnp.int32)]).reshape(NW, NCHUNK, CHUNK)
    dst_p = jnp.concatenate(
        [edge_index[1], jnp.zeros((pad,), jnp.int32)]).reshape(NW, NCHUNK, CHUNK)
    sd_p = jnp.stack([src_p, dst_p], axis=2)
    w_p = jnp.concatenate(
        [edge_weight, jnp.zeros((pad,), jnp.float32)]).reshape(NW, NCHUNK, CHUNK)


    be = b_emb.reshape(1, NHID)
    bih = b_ih.reshape(1, 3 * NHID)
    bhh = b_hh.reshape(1, 3 * NHID)
    bm = b_mlp.reshape(1, NCLASS)

    x_in, m = _embed_conv(h, W_emb, be, conv_w[0])
    p = _segsum(m, sd_p, w_p, 'segsum_l1')
    x1, m1 = _gru_conv(p, x_in, W_ih, bih, W_hh, bhh, conv_w[1])
    p1 = _segsum(m1, sd_p, w_p, 'segsum_l2')
    return _gru_head(p1, x1, x_in, W_ih, bih, W_hh, bhh, W_mlp, bm)


# speedup vs baseline: 1.2847x; 1.2847x over previous
import jax
import jax.numpy as jnp
from jax import lax
from jax.experimental import pallas as pl
from jax.experimental.pallas import tpu as pltpu
from jax.experimental.pallas import tpu_sc as plsc

N = 10000
E = 320000
NHID = 128
NCLASS = 40

NC = 2
NS = 16
NW = NC * NS
CHUNK = 128
NCHUNK = -(-E // (NW * CHUNK))
E_PAD = NW * NCHUNK * CHUNK
ROWS_PER_TILE = 624
TAIL_BASE = NS * ROWS_PER_TILE
TAIL_ROWS = N - TAIL_BASE

LANES = 16
FB = NHID // LANES


def _segsum_body(m_hbm, src_hbm, dst_hbm, w_hbm, out_hbm,
                 acc, src_v, dst_v, w_v, rows, gsem):
    c = lax.axis_index("c")
    s = lax.axis_index("s")
    wid = c * NS + s

    def _zrow(r, _):
        for f in range(FB):
            rows[r, pl.ds(f * LANES, LANES)] = jnp.zeros((LANES,), jnp.float32)
        return 0
    lax.fori_loop(0, CHUNK, _zrow, 0)
    base = s * ROWS_PER_TILE
    off = 0
    while off < ROWS_PER_TILE:
        nrow = min(CHUNK, ROWS_PER_TILE - off)
        pltpu.sync_copy(rows.at[pl.ds(0, nrow)], acc.at[pl.ds(base + off, nrow)])
        off += nrow

    @pl.when(s == 0)
    def _zero_tail():
        pltpu.sync_copy(rows.at[pl.ds(0, TAIL_ROWS)],
                        acc.at[pl.ds(TAIL_BASE, TAIL_ROWS)])

    pltpu.sync_copy(src_hbm.at[wid], src_v)
    pltpu.sync_copy(dst_hbm.at[wid], dst_v)
    pltpu.sync_copy(w_hbm.at[wid], w_v)

    plsc.subcore_barrier()

    def _chunk(j, _):
        pltpu.async_copy(m_hbm.at[src_v.at[j]], rows, gsem).wait()

        def _scale(g, _):
            w16 = w_v[j, pl.ds(g * LANES, LANES)]
            for u in range(LANES):
                e = g * LANES + u
                wb = jnp.full((LANES,), w16[u], jnp.float32)
                for f in range(FB):
                    sl = pl.ds(f * LANES, LANES)
                    rows[e, sl] = rows[e, sl] * wb
            return 0
        lax.fori_loop(0, CHUNK // LANES, _scale, 0)

        pltpu.sync_copy(rows, acc.at[dst_v.at[j]], add=True)
        return 0
    lax.fori_loop(0, NCHUNK, _chunk, 0)

    plsc.subcore_barrier()

    pltpu.sync_copy(acc.at[pl.ds(base, ROWS_PER_TILE)],
                    out_hbm.at[c, pl.ds(base, ROWS_PER_TILE)])

    @pl.when(s == 0)
    def _out_tail():
        pltpu.sync_copy(acc.at[pl.ds(TAIL_BASE, TAIL_ROWS)],
                        out_hbm.at[c, pl.ds(TAIL_BASE, TAIL_ROWS)])


def _segsum(m, src_p, dst_p, w_p, name):
    mesh = plsc.VectorSubcoreMesh(core_axis_name="c", subcore_axis_name="s")
    return pl.kernel(
        _segsum_body,
        out_type=jax.ShapeDtypeStruct((NC, N, NHID), jnp.float32),
        mesh=mesh,
        name=name,
        scratch_types=[
            pltpu.VMEM_SHARED((N, NHID), jnp.float32),
            pltpu.VMEM((NCHUNK, CHUNK), jnp.int32),
            pltpu.VMEM((NCHUNK, CHUNK), jnp.int32),
            pltpu.VMEM((NCHUNK, CHUNK), jnp.float32),
            pltpu.VMEM((CHUNK, NHID), jnp.float32),
            pltpu.SemaphoreType.DMA,
        ],
    )(m, src_p, dst_p, w_p)


ROWS_BLK = 1000
GRID = N // ROWS_BLK


def _dot_t(a, b):
    return lax.dot_general(a, b, (((1,), (1,)), ((), ())),
                           preferred_element_type=jnp.float32)


def _dot(a, b):
    return lax.dot_general(a, b, (((1,), (0,)), ((), ())),
                           preferred_element_type=jnp.float32)


def _gru(agg, x, W_ih, b_ih, W_hh, b_hh):
    gi = _dot_t(agg, W_ih) + b_ih
    gh = _dot_t(x, W_hh) + b_hh
    r = jax.nn.sigmoid(gi[:, 0:NHID] + gh[:, 0:NHID])
    z = jax.nn.sigmoid(gi[:, NHID:2 * NHID] + gh[:, NHID:2 * NHID])
    n = jnp.tanh(gi[:, 2 * NHID:] + r * gh[:, 2 * NHID:])
    return (1.0 - z) * n + z * x


def _embed_conv_body(h_ref, We_ref, be_ref, cw_ref, x_ref, m_ref):
    x = _dot_t(h_ref[...], We_ref[...]) + be_ref[...]
    x_ref[...] = x
    m_ref[...] = _dot(x, cw_ref[...])


def _embed_conv(h, W_emb, b_emb, cw0):
    blk = lambda i: (i, 0)
    full = lambda i: (0, 0)
    return pl.pallas_call(
        _embed_conv_body,
        grid=(GRID,),
        in_specs=[
            pl.BlockSpec((ROWS_BLK, NHID), blk),
            pl.BlockSpec((NHID, NHID), full),
            pl.BlockSpec((1, NHID), full),
            pl.BlockSpec((NHID, NHID), full),
        ],
        out_specs=[pl.BlockSpec((ROWS_BLK, NHID), blk),
                   pl.BlockSpec((ROWS_BLK, NHID), blk)],
        out_shape=[jax.ShapeDtypeStruct((N, NHID), jnp.float32),
                   jax.ShapeDtypeStruct((N, NHID), jnp.float32)],
    )(h, W_emb, b_emb, cw0)


def _gru_conv_body(p_ref, x_ref, Wih_ref, bih_ref, Whh_ref, bhh_ref, cw_ref,
                   xo_ref, mo_ref):
    agg = p_ref[0] + p_ref[1]
    xn = _gru(agg, x_ref[...], Wih_ref[...], bih_ref[...],
              Whh_ref[...], bhh_ref[...])
    xo_ref[...] = xn
    mo_ref[...] = _dot(xn, cw_ref[...])


def _gru_conv(p, x, W_ih, b_ih, W_hh, b_hh, cw1):
    blk = lambda i: (i, 0)
    full = lambda i: (0, 0)
    return pl.pallas_call(
        _gru_conv_body,
        grid=(GRID,),
        in_specs=[
            pl.BlockSpec((NC, ROWS_BLK, NHID), lambda i: (0, i, 0)),
            pl.BlockSpec((ROWS_BLK, NHID), blk),
            pl.BlockSpec((3 * NHID, NHID), full),
            pl.BlockSpec((1, 3 * NHID), full),
            pl.BlockSpec((3 * NHID, NHID), full),
            pl.BlockSpec((1, 3 * NHID), full),
            pl.BlockSpec((NHID, NHID), full),
        ],
        out_specs=[pl.BlockSpec((ROWS_BLK, NHID), blk),
                   pl.BlockSpec((ROWS_BLK, NHID), blk)],
        out_shape=[jax.ShapeDtypeStruct((N, NHID), jnp.float32),
                   jax.ShapeDtypeStruct((N, NHID), jnp.float32)],
    )(p, x, W_ih, b_ih, W_hh, b_hh, cw1)


def _gru_head_body(p_ref, x_ref, xin_ref, Wih_ref, bih_ref, Whh_ref, bhh_ref,
                   Wm_ref, bm_ref, o_ref):
    agg = p_ref[0] + p_ref[1]
    xn = _gru(agg, x_ref[...], Wih_ref[...], bih_ref[...],
              Whh_ref[...], bhh_ref[...])
    xr = xin_ref[...] + xn
    logits = _dot_t(xr, Wm_ref[...]) + bm_ref[...]
    shifted = logits - jnp.max(logits, axis=1, keepdims=True)
    o_ref[...] = shifted - jnp.log(
        jnp.sum(jnp.exp(shifted), axis=1, keepdims=True))


def _gru_head(p, x, x_in, W_ih, b_ih, W_hh, b_hh, W_mlp, b_mlp):
    blk = lambda i: (i, 0)
    full = lambda i: (0, 0)
    return pl.pallas_call(
        _gru_head_body,
        grid=(GRID,),
        in_specs=[
            pl.BlockSpec((NC, ROWS_BLK, NHID), lambda i: (0, i, 0)),
            pl.BlockSpec((ROWS_BLK, NHID), blk),
            pl.BlockSpec((ROWS_BLK, NHID), blk),
            pl.BlockSpec((3 * NHID, NHID), full),
            pl.BlockSpec((1, 3 * NHID), full),
            pl.BlockSpec((3 * NHID, NHID), full),
            pl.BlockSpec((1, 3 * NHID), full),
            pl.BlockSpec((NCLASS, NHID), full),
            pl.BlockSpec((1, NCLASS), full),
        ],
        out_specs=pl.BlockSpec((ROWS_BLK, NCLASS), blk),
        out_shape=jax.ShapeDtypeStruct((N, NCLASS), jnp.float32),
    )(p, x, x_in, W_ih, b_ih, W_hh, b_hh, W_mlp, b_mlp)


def kernel(h, edge_index, edge_weight, W_emb, b_emb, conv_w,
           W_ih, b_ih, W_hh, b_hh, W_mlp, b_mlp):
    pad = E_PAD - E
    src_p = jnp.concatenate(
        [edge_index[0], jnp.zeros((pad,), jnp.int32)]).reshape(NW, NCHUNK, CHUNK)
    dst_p = jnp.concatenate(
        [edge_index[1], jnp.zeros((pad,), jnp.int32)]).reshape(NW, NCHUNK, CHUNK)
    w_p = jnp.concatenate(
        [edge_weight, jnp.zeros((pad,), jnp.float32)]).reshape(NW, NCHUNK, CHUNK)


    be = b_emb.reshape(1, NHID)
    bih = b_ih.reshape(1, 3 * NHID)
    bhh = b_hh.reshape(1, 3 * NHID)
    bm = b_mlp.reshape(1, NCLASS)

    x_in, m = _embed_conv(h, W_emb, be, conv_w[0])
    p = _segsum(m, src_p, dst_p, w_p, 'segsum_l1')
    x1, m1 = _gru_conv(p, x_in, W_ih, bih, W_hh, bhh, conv_w[1])
    p1 = _segsum(m1, src_p, dst_p, w_p, 'segsum_l2')
    return _gru_head(p1, x1, x_in, W_ih, bih, W_hh, bhh, W_mlp, bm)
